# baseline (device time: 109576 ns/iter reference)
import jax
import jax.numpy as jnp
from jax import lax
from jax.experimental import pallas as pl
from jax.experimental.pallas import tpu as pltpu

N_DEV = 4
N_EXP = 32
CAP = 51
SLOTS = 64


def kernel(x, router_W, route_idx, expert_W):
    n_tok, d = x.shape
    e_local, _, h = expert_W.shape
    chunk = n_tok // N_DEV
    n_slots = e_local * SLOTS

    e = route_idx[:, 0].astype(jnp.int32)
    onehot = (e[:, None] == jnp.arange(N_EXP, dtype=jnp.int32)[None, :]).astype(
        jnp.int32
    )
    pos = jnp.cumsum(onehot, axis=0) - onehot
    myslot = jnp.sum(pos * onehot, axis=1)
    keep = myslot < CAP
    tok = jnp.arange(n_tok, dtype=jnp.int32)
    flat = jnp.where(keep, e * SLOTS + myslot, N_EXP * SLOTS)
    idx_all = (
        jnp.full((N_EXP * SLOTS + 1,), n_tok, jnp.int32).at[flat].set(tok)
    )[: N_EXP * SLOTS].reshape(N_EXP, SLOTS)

    my = lax.axis_index("i")
    idx_local = lax.dynamic_slice_in_dim(idx_all, my * e_local, e_local, 0)
    idx_vec = idx_local.reshape(1, n_slots)
    xg = jnp.take(
        x, jnp.clip(idx_local.reshape(-1), 0, n_tok - 1), axis=0
    )

    def body(xg_ref, w_ref, idx_ref, out_ref, y_ref, send_ref, recv_ref,
             send_sems, recv_sems):
        my_pos = lax.axis_index("i")
        left = (my_pos - 1) % N_DEV
        right = (my_pos + 1) % N_DEV

        barrier_sem = pltpu.get_barrier_semaphore()
        for nbr in [left, right]:
            pl.semaphore_signal(
                barrier_sem, inc=1,
                device_id=(nbr,), device_id_type=pl.DeviceIdType.MESH,
            )
        pl.semaphore_wait(barrier_sem, 2)

        for j in range(e_local):
            y_ref[j * SLOTS:(j + 1) * SLOTS, :] = jnp.dot(
                xg_ref[j * SLOTS:(j + 1) * SLOTS, :],
                w_ref[j],
                preferred_element_type=jnp.float32,
            )

        def scat_chunk(c):
            rows = c * chunk + lax.broadcasted_iota(
                jnp.int32, (chunk, n_slots), 0
            )
            s_mat = (rows == idx_ref[:, :]).astype(jnp.float32)
            return jnp.dot(s_mat, y_ref[:, :], preferred_element_type=jnp.float32)

        acc = scat_chunk((my_pos - 1) % N_DEV)
        for t in range(N_DEV - 1):
            send_ref[t] = acc
            rdma = pltpu.make_async_remote_copy(
                src_ref=send_ref.at[t],
                dst_ref=recv_ref.at[t],
                send_sem=send_sems.at[t],
                recv_sem=recv_sems.at[t],
                device_id=(right,),
                device_id_type=pl.DeviceIdType.MESH,
            )
            rdma.start()
            rdma.wait()
            acc = recv_ref[t] + scat_chunk((my_pos - 2 - t) % N_DEV)
        out_ref[:, :] = acc

    return pl.pallas_call(
        body,
        out_shape=jax.ShapeDtypeStruct((chunk, h), jnp.float32),
        in_specs=[
            pl.BlockSpec(memory_space=pltpu.VMEM),
            pl.BlockSpec(memory_space=pltpu.VMEM),
            pl.BlockSpec(memory_space=pltpu.VMEM),
        ],
        out_specs=pl.BlockSpec(memory_space=pltpu.VMEM),
        scratch_shapes=[
            pltpu.VMEM((n_slots, h), jnp.float32),
            pltpu.VMEM((N_DEV - 1, chunk, h), jnp.float32),
            pltpu.VMEM((N_DEV - 1, chunk, h), jnp.float32),
            pltpu.SemaphoreType.DMA((N_DEV - 1,)),
            pltpu.SemaphoreType.DMA((N_DEV - 1,)),
        ],
        compiler_params=pltpu.CompilerParams(collective_id=0),
    )(xg, expert_W, idx_vec)


# device time: 60346 ns/iter; 1.8158x vs baseline; 1.8158x over previous
import jax
import jax.numpy as jnp
from jax import lax
from jax.experimental import pallas as pl
from jax.experimental.pallas import tpu as pltpu

N_DEV = 4
N_EXP = 32
CAP = 51
SLOTS = 64


def kernel(x, router_W, route_idx, expert_W):
    n_tok, d = x.shape
    e_local, _, h = expert_W.shape
    chunk = n_tok // N_DEV
    n_slots = e_local * SLOTS

    e = route_idx[:, 0].astype(jnp.int32)
    onehot = (e[:, None] == jnp.arange(N_EXP, dtype=jnp.int32)[None, :]).astype(
        jnp.int32
    )
    pos = jnp.cumsum(onehot, axis=0) - onehot
    myslot = jnp.sum(pos * onehot, axis=1)
    keep = myslot < CAP
    tok = jnp.arange(n_tok, dtype=jnp.int32)
    flat = jnp.where(keep, e * SLOTS + myslot, N_EXP * SLOTS)
    idx_all = (
        jnp.full((N_EXP * SLOTS + 1,), n_tok, jnp.int32).at[flat].set(tok)
    )[: N_EXP * SLOTS].reshape(N_EXP, SLOTS)

    my = lax.axis_index("i")
    idx_local = lax.dynamic_slice_in_dim(idx_all, my * e_local, e_local, 0)
    idx_vec = idx_local.reshape(1, n_slots)
    xg = jnp.take(
        x, jnp.clip(idx_local.reshape(-1), 0, n_tok - 1), axis=0
    )

    def body(xg_ref, w_ref, idx_ref, out_ref, y_ref, send_ref, recv_ref,
             send_sems, recv_sems):
        my_pos = lax.axis_index("i")

        barrier_sem = pltpu.get_barrier_semaphore()
        for k in range(1, N_DEV):
            pl.semaphore_signal(
                barrier_sem, inc=1,
                device_id=((my_pos + k) % N_DEV,),
                device_id_type=pl.DeviceIdType.MESH,
            )
        pl.semaphore_wait(barrier_sem, N_DEV - 1)

        for j in range(e_local):
            y_ref[j * SLOTS:(j + 1) * SLOTS, :] = jnp.dot(
                xg_ref[j * SLOTS:(j + 1) * SLOTS, :].astype(jnp.bfloat16),
                w_ref[j].astype(jnp.bfloat16),
                preferred_element_type=jnp.float32,
            ).astype(jnp.bfloat16)

        def scat_chunk(c, out_dtype):
            rows = c * chunk + lax.broadcasted_iota(
                jnp.int32, (chunk, n_slots), 0
            )
            s_mat = (rows == idx_ref[:, :]).astype(jnp.bfloat16)
            return jnp.dot(
                s_mat, y_ref[:, :], preferred_element_type=jnp.float32
            ).astype(out_dtype)

        rdmas = []
        for k in range(1, N_DEV):
            target = (my_pos + k) % N_DEV
            send_ref[k - 1] = scat_chunk(target, jnp.bfloat16)
            rdma = pltpu.make_async_remote_copy(
                src_ref=send_ref.at[k - 1],
                dst_ref=recv_ref.at[N_DEV - 1 - k],
                send_sem=send_sems.at[k - 1],
                recv_sem=recv_sems.at[N_DEV - 1 - k],
                device_id=(target,),
                device_id_type=pl.DeviceIdType.MESH,
            )
            rdma.start()
            rdmas.append(rdma)

        acc = scat_chunk(my_pos, jnp.float32)
        for k in range(1, N_DEV):
            rdmas[k - 1].wait_recv()
            acc = acc + recv_ref[N_DEV - 1 - k].astype(jnp.float32)
        for r in rdmas:
            r.wait_send()
        out_ref[:, :] = acc

    return pl.pallas_call(
        body,
        out_shape=jax.ShapeDtypeStruct((chunk, h), jnp.float32),
        in_specs=[
            pl.BlockSpec(memory_space=pltpu.VMEM),
            pl.BlockSpec(memory_space=pltpu.VMEM),
            pl.BlockSpec(memory_space=pltpu.VMEM),
        ],
        out_specs=pl.BlockSpec(memory_space=pltpu.VMEM),
        scratch_shapes=[
            pltpu.VMEM((n_slots, h), jnp.bfloat16),
            pltpu.VMEM((N_DEV - 1, chunk, h), jnp.bfloat16),
            pltpu.VMEM((N_DEV - 1, chunk, h), jnp.bfloat16),
            pltpu.SemaphoreType.DMA((N_DEV - 1,)),
            pltpu.SemaphoreType.DMA((N_DEV - 1,)),
        ],
        compiler_params=pltpu.CompilerParams(collective_id=0),
    )(xg, expert_W, idx_vec)


# device time: 48981 ns/iter; 2.2371x vs baseline; 1.2320x over previous
import jax
import jax.numpy as jnp
from jax import lax
from jax.experimental import pallas as pl
from jax.experimental.pallas import tpu as pltpu

N_DEV = 4
N_EXP = 32
CAP = 51
SLOTS = 64


def kernel(x, router_W, route_idx, expert_W):
    n_tok, d = x.shape
    e_local, _, h = expert_W.shape
    chunk = n_tok // N_DEV
    n_slots = e_local * SLOTS

    def body(x_ref, e_ref, w_ref, out_ref, gt_ref, y_ref, send_ref, recv_ref,
             send_sems, recv_sems):
        my_pos = lax.axis_index("i")

        barrier_sem = pltpu.get_barrier_semaphore()
        for k in range(1, N_DEV):
            pl.semaphore_signal(
                barrier_sem, inc=1,
                device_id=((my_pos + k) % N_DEV,),
                device_id_type=pl.DeviceIdType.MESH,
            )
        pl.semaphore_wait(barrier_sem, N_DEV - 1)

        e_col = e_ref[:, :]
        onehot = (
            e_col == lax.broadcasted_iota(jnp.int32, (1, N_EXP), 1)
        ).astype(jnp.bfloat16)
        tri = (
            lax.broadcasted_iota(jnp.int32, (n_tok, n_tok), 0)
            > lax.broadcasted_iota(jnp.int32, (n_tok, n_tok), 1)
        ).astype(jnp.bfloat16)
        pos = jnp.dot(tri, onehot, preferred_element_type=jnp.float32)
        pos_tok = jnp.sum(
            pos * onehot.astype(jnp.float32), axis=1, keepdims=True
        )

        slot_iota = lax.broadcasted_iota(jnp.int32, (1, n_slots), 1)
        slot_exp = my_pos * e_local + slot_iota // SLOTS
        slot_pos = (slot_iota % SLOTS).astype(jnp.float32)
        gt = (
            (e_col == slot_exp)
            & (pos_tok == slot_pos)
            & (pos_tok < float(CAP))
        ).astype(jnp.bfloat16)
        gt_ref[:, :] = gt

        xg = lax.dot_general(
            gt, x_ref[:, :].astype(jnp.bfloat16),
            (((0,), (0,)), ((), ())),
            preferred_element_type=jnp.float32,
        ).astype(jnp.bfloat16)
        for j in range(e_local):
            y_ref[j * SLOTS:(j + 1) * SLOTS, :] = jnp.dot(
                xg[j * SLOTS:(j + 1) * SLOTS, :],
                w_ref[j].astype(jnp.bfloat16),
                preferred_element_type=jnp.float32,
            ).astype(jnp.bfloat16)

        def scat_chunk(c, out_dtype):
            s_mat = gt_ref[pl.ds(c * chunk, chunk), :]
            return jnp.dot(
                s_mat, y_ref[:, :], preferred_element_type=jnp.float32
            ).astype(out_dtype)

        rdmas = []
        for k in range(1, N_DEV):
            target = (my_pos + k) % N_DEV
            send_ref[k - 1] = scat_chunk(target, jnp.bfloat16)
            rdma = pltpu.make_async_remote_copy(
                src_ref=send_ref.at[k - 1],
                dst_ref=recv_ref.at[N_DEV - 1 - k],
                send_sem=send_sems.at[k - 1],
                recv_sem=recv_sems.at[N_DEV - 1 - k],
                device_id=(target,),
                device_id_type=pl.DeviceIdType.MESH,
            )
            rdma.start()
            rdmas.append(rdma)

        acc = scat_chunk(my_pos, jnp.float32)
        for k in range(1, N_DEV):
            rdmas[k - 1].wait_recv()
            acc = acc + recv_ref[N_DEV - 1 - k].astype(jnp.float32)
        for r in rdmas:
            r.wait_send()
        out_ref[:, :] = acc

    return pl.pallas_call(
        body,
        out_shape=jax.ShapeDtypeStruct((chunk, h), jnp.float32),
        in_specs=[
            pl.BlockSpec(memory_space=pltpu.VMEM),
            pl.BlockSpec(memory_space=pltpu.VMEM),
            pl.BlockSpec(memory_space=pltpu.VMEM),
        ],
        out_specs=pl.BlockSpec(memory_space=pltpu.VMEM),
        scratch_shapes=[
            pltpu.VMEM((n_tok, n_slots), jnp.bfloat16),
            pltpu.VMEM((n_slots, h), jnp.bfloat16),
            pltpu.VMEM((N_DEV - 1, chunk, h), jnp.bfloat16),
            pltpu.VMEM((N_DEV - 1, chunk, h), jnp.bfloat16),
            pltpu.SemaphoreType.DMA((N_DEV - 1,)),
            pltpu.SemaphoreType.DMA((N_DEV - 1,)),
        ],
        compiler_params=pltpu.CompilerParams(collective_id=0),
    )(x, route_idx, expert_W)


# device time: 44631 ns/iter; 2.4552x vs baseline; 1.0975x over previous
import jax
import jax.numpy as jnp
from jax import lax
from jax.experimental import pallas as pl
from jax.experimental.pallas import tpu as pltpu

N_DEV = 4
N_EXP = 32
CAP = 51
SLOTS = 64


def kernel(x, router_W, route_idx, expert_W):
    n_tok, d = x.shape
    e_local, _, h = expert_W.shape
    chunk = n_tok // N_DEV
    n_slots = e_local * SLOTS

    def body(x_ref, e_ref, w_ref, out_ref, gt_ref, wv_ref, y_ref, send_ref,
             recv_ref, w_sems, send_sems, recv_sems):
        my_pos = lax.axis_index("i")

        w_dmas = []
        for j in range(e_local):
            dma = pltpu.make_async_copy(w_ref.at[j], wv_ref.at[j], w_sems.at[j])
            dma.start()
            w_dmas.append(dma)

        barrier_sem = pltpu.get_barrier_semaphore()
        for k in range(1, N_DEV):
            pl.semaphore_signal(
                barrier_sem, inc=1,
                device_id=((my_pos + k) % N_DEV,),
                device_id_type=pl.DeviceIdType.MESH,
            )
        pl.semaphore_wait(barrier_sem, N_DEV - 1)

        e_col = e_ref[:, :]
        onehot = (
            e_col == lax.broadcasted_iota(jnp.int32, (1, N_EXP), 1)
        ).astype(jnp.bfloat16)
        tri = (
            lax.broadcasted_iota(jnp.int32, (n_tok, n_tok), 0)
            > lax.broadcasted_iota(jnp.int32, (n_tok, n_tok), 1)
        ).astype(jnp.bfloat16)
        pos = jnp.dot(tri, onehot, preferred_element_type=jnp.float32)
        pos_tok = jnp.sum(
            pos * onehot.astype(jnp.float32), axis=1, keepdims=True
        )

        slot_iota = lax.broadcasted_iota(jnp.int32, (1, n_slots), 1)
        slot_exp = my_pos * e_local + slot_iota // SLOTS
        slot_pos = (slot_iota % SLOTS).astype(jnp.float32)
        gt = (
            (e_col == slot_exp)
            & (pos_tok == slot_pos)
            & (pos_tok < float(CAP))
        ).astype(jnp.bfloat16)
        gt_ref[:, :] = gt

        xg = lax.dot_general(
            gt, x_ref[:, :].astype(jnp.bfloat16),
            (((0,), (0,)), ((), ())),
            preferred_element_type=jnp.float32,
        ).astype(jnp.bfloat16)
        for j in range(e_local):
            w_dmas[j].wait()
            y_ref[j * SLOTS:(j + 1) * SLOTS, :] = jnp.dot(
                xg[j * SLOTS:(j + 1) * SLOTS, :],
                wv_ref[j].astype(jnp.bfloat16),
                preferred_element_type=jnp.float32,
            ).astype(jnp.bfloat16)

        def scat_chunk(c, out_dtype):
            s_mat = gt_ref[pl.ds(c * chunk, chunk), :]
            return jnp.dot(
                s_mat, y_ref[:, :], preferred_element_type=jnp.float32
            ).astype(out_dtype)

        rdmas = []
        for k in range(1, N_DEV):
            target = (my_pos + k) % N_DEV
            send_ref[k - 1] = scat_chunk(target, jnp.bfloat16)
            rdma = pltpu.make_async_remote_copy(
                src_ref=send_ref.at[k - 1],
                dst_ref=recv_ref.at[N_DEV - 1 - k],
                send_sem=send_sems.at[k - 1],
                recv_sem=recv_sems.at[N_DEV - 1 - k],
                device_id=(target,),
                device_id_type=pl.DeviceIdType.MESH,
            )
            rdma.start()
            rdmas.append(rdma)

        acc = scat_chunk(my_pos, jnp.float32)
        for k in range(1, N_DEV):
            rdmas[k - 1].wait_recv()
            acc = acc + recv_ref[N_DEV - 1 - k].astype(jnp.float32)
        for r in rdmas:
            r.wait_send()
        out_ref[:, :] = acc

    return pl.pallas_call(
        body,
        out_shape=jax.ShapeDtypeStruct((chunk, h), jnp.float32),
        in_specs=[
            pl.BlockSpec(memory_space=pltpu.VMEM),
            pl.BlockSpec(memory_space=pltpu.VMEM),
            pl.BlockSpec(memory_space=pltpu.MemorySpace.HBM),
        ],
        out_specs=pl.BlockSpec(memory_space=pltpu.VMEM),
        scratch_shapes=[
            pltpu.VMEM((n_tok, n_slots), jnp.bfloat16),
            pltpu.VMEM((e_local, d, h), jnp.float32),
            pltpu.VMEM((n_slots, h), jnp.bfloat16),
            pltpu.VMEM((N_DEV - 1, chunk, h), jnp.bfloat16),
            pltpu.VMEM((N_DEV - 1, chunk, h), jnp.bfloat16),
            pltpu.SemaphoreType.DMA((e_local,)),
            pltpu.SemaphoreType.DMA((N_DEV - 1,)),
            pltpu.SemaphoreType.DMA((N_DEV - 1,)),
        ],
        compiler_params=pltpu.CompilerParams(collective_id=0),
    )(x, route_idx, expert_W)
